# baseline (speedup 1.0000x reference)
import functools

import jax
import jax.numpy as jnp
from jax import lax
from jax.experimental import pallas as pl
from jax.experimental.pallas import tpu as pltpu
from jax.experimental.pallas import tpu_sc as plsc

_L = 16
_C = 32
_RPB = 16


def _bprmf_kernel(B, D, n_workers):
    bpw = B // n_workers
    n_chunks = bpw // _C
    mesh = plsc.VectorSubcoreMesh(core_axis_name="c", subcore_axis_name="s")

    @functools.partial(
        pl.kernel,
        mesh=mesh,
        compiler_params=pltpu.CompilerParams(needs_layout_passes=False),
        out_type=jax.ShapeDtypeStruct((B,), jnp.float32),
        scratch_types=[
            pltpu.VMEM((bpw,), jnp.int32),
            pltpu.VMEM((bpw,), jnp.int32),
            pltpu.VMEM((_C,), jnp.int32),
            pltpu.VMEM((_C,), jnp.int32),
            pltpu.VMEM((_C, _RPB * 64 // 128, 128), jnp.float32),
            pltpu.VMEM((_C, _RPB * 64 // 128, 128), jnp.float32),
            pltpu.VMEM((bpw,), jnp.float32),
            pltpu.VMEM((bpw,), jnp.float32),
            pltpu.SemaphoreType.DMA,
            pltpu.SemaphoreType.DMA,
            pltpu.SemaphoreType.DMA,
        ],
    )
    def run(u_hbm, i_hbm, p_hbm, q_hbm, b_hbm, out_hbm,
            uv, iv, pidx, qidx, pv, qv, bv, ov, semb, semp, semq):
        wid = lax.axis_index("s") * 2 + lax.axis_index("c")
        base = wid * bpw
        pltpu.sync_copy(u_hbm.at[pl.ds(base, bpw)], uv)
        pltpu.sync_copy(i_hbm.at[pl.ds(base, bpw)], iv)
        cp_b = pltpu.async_copy(b_hbm.at[iv], bv, semb)

        lanes = lax.iota(jnp.int32, _L)

        def chunk_body(ch, _):
            c0 = ch * _C
            for g in range(_C // _L):
                usl = uv[pl.ds(c0 + g * _L, _L)]
                isl = iv[pl.ds(c0 + g * _L, _L)]
                pidx[pl.ds(g * _L, _L)] = usl >> 4
                qidx[pl.ds(g * _L, _L)] = isl >> 4
            cp_p = pltpu.async_copy(p_hbm.at[pidx], pv, semp)
            cp_q = pltpu.async_copy(q_hbm.at[qidx], qv, semq)
            cp_p.wait()
            cp_q.wait()
            for g in range(_C // _L):
                usl = uv[pl.ds(c0 + g * _L, _L)]
                isl = iv[pl.ds(c0 + g * _L, _L)]
                kvec = g * _L + lanes
                pw = (usl & 15) * D
                qw = (isl & 15) * D
                acc = jnp.zeros((_L,), jnp.float32)
                for c in range(D):
                    pwc = pw + c
                    qwc = qw + c
                    pcol = plsc.load_gather(pv, [kvec, pwc >> 7, pwc & 127])
                    qcol = plsc.load_gather(qv, [kvec, qwc >> 7, qwc & 127])
                    acc = acc + pcol * qcol
                ov[pl.ds(c0 + g * _L, _L)] = acc
            return 0

        lax.fori_loop(0, n_chunks, chunk_body, 0)
        cp_b.wait()

        def bias_body(g, _):
            sl = pl.ds(g * _L, _L)
            ov[sl] = ov[sl] + bv[sl]
            return 0

        lax.fori_loop(0, bpw // _L, bias_body, 0)
        pltpu.sync_copy(ov, out_hbm.at[pl.ds(base, bpw)])

    return run


def kernel(u, i, P, Q, bi):
    B = u.shape[0]
    N, D = P.shape
    P3 = P.reshape(N // _RPB, _RPB * D // 128, 128)
    Q3 = Q.reshape(N // _RPB, _RPB * D // 128, 128)
    return _bprmf_kernel(B, D, 32)(u, i, P3, Q3, bi.reshape(-1))
